# baseline (speedup 1.0000x reference)
import functools

import jax
import jax.numpy as jnp
import numpy as np
from jax import lax
from jax.experimental import pallas as pl
from jax.experimental.pallas import tpu as pltpu
from jax.experimental.pallas import tpu_sc as plsc

N = 10000
E = 320000
D = 128
IW = 64
NC = 2
NS = 16
NW = NC * NS
WPB = 160
SROWS = NW * WPB
CHUNK = 32
NCHUNK = WPB // CHUNK
WPB0 = 288
WPB1 = 32
SUBR = 640
ACC_N = NS * SUBR
DEGW = 128
DW = D // 2

_T_IDX = np.zeros(D, dtype=np.int32)
for _g in range(4):
    for _i in range(16):
        _T_IDX[_g * 32 + _i] = _g * 32 + 2 * _i
        _T_IDX[_g * 32 + 16 + _i] = _g * 32 + 2 * _i + 1


def _sc_agg_body(table, src2, dst2, agg_out, acc_sh, src_slab, dst_slab,
                 bf0, bf1, rows_f, sem_g0, sem_g1):
    c = lax.axis_index("c")
    s = lax.axis_index("s")

    def z_rows(k, _):
        rows_f[k // 8, pl.ds((k % 8) * 16, 16)] = jnp.zeros((16,), jnp.float32)
        return _
    lax.fori_loop(0, IW * 8, z_rows, None)
    for j in range(SUBR // IW):
        pltpu.sync_copy(rows_f, acc_sh.at[pl.ds(s * SUBR + j * IW, IW)])
    plsc.subcore_barrier()

    def gather(r, buf, sem):
        return pltpu.async_copy(table.at[src_slab.at[r]], buf, sem)

    def wait_gather(buf, sem):
        pltpu.make_async_copy(table.at[pl.ds(0, IW)], buf, sem).wait()

    def convert(buf):
        def cv(r, _):
            sh16 = jnp.full((16,), 16, jnp.int32)
            hi = jnp.full((16,), -65536, jnp.int32)
            for g in range(4):
                w_ = buf[r, pl.ds(g * 16, 16)]
                a = lax.bitcast_convert_type(lax.shift_left(w_, sh16),
                                             jnp.float32)
                b = lax.bitcast_convert_type(lax.bitwise_and(w_, hi),
                                             jnp.float32)
                rows_f[r, pl.ds(g * 32, 16)] = a
                rows_f[r, pl.ds(g * 32 + 16, 16)] = b
            return _
        lax.fori_loop(0, IW, cv, None)

    def scatter(r):
        pltpu.sync_copy(rows_f, acc_sh.at[dst_slab.at[r]], add=True)

    def run_chunks(base_rows, nchunk):
        for ci in range(nchunk):
            pltpu.sync_copy(src2.at[pl.ds(base_rows + ci * CHUNK, CHUNK)],
                            src_slab)
            pltpu.sync_copy(dst2.at[pl.ds(base_rows + ci * CHUNK, CHUNK)],
                            dst_slab)
            gather(0, bf0, sem_g0)

            def pair(p, _):
                wait_gather(bf0, sem_g0)
                gather(2 * p + 1, bf1, sem_g1)
                convert(bf0)
                scatter(2 * p)
                wait_gather(bf1, sem_g1)

                @pl.when(p < CHUNK // 2 - 1)
                def _():
                    gather(2 * p + 2, bf0, sem_g0)
                convert(bf1)
                scatter(2 * p + 1)
                return _
            lax.fori_loop(0, CHUNK // 2, pair, None)

    @pl.when(c == 0)
    def _():
        run_chunks(s * WPB0, WPB0 // CHUNK)

    if WPB1:
        @pl.when(c == 1)
        def _():
            run_chunks(NS * WPB0 + s * WPB1, WPB1 // CHUNK)
    plsc.subcore_barrier()

    pltpu.sync_copy(acc_sh.at[pl.ds(s * SUBR, SUBR)],
                    agg_out.at[c].at[pl.ds(s * SUBR, SUBR)])


@functools.lru_cache(maxsize=None)
def _sc_agg_kernel():
    return pl.kernel(
        _sc_agg_body,
        out_type=jax.ShapeDtypeStruct((NC, ACC_N, D), jnp.float32),
        mesh=plsc.VectorSubcoreMesh(core_axis_name="c", subcore_axis_name="s"),
        compiler_params=pltpu.CompilerParams(use_tc_tiling_on_sc=False),
        scratch_types=[
            pltpu.VMEM_SHARED((ACC_N, D), jnp.float32),
            pltpu.VMEM((CHUNK, IW), jnp.int32),
            pltpu.VMEM((CHUNK, IW), jnp.int32),
            pltpu.VMEM((IW, DW), jnp.int32),
            pltpu.VMEM((IW, DW), jnp.int32),
            pltpu.VMEM((IW, D), jnp.float32),
            pltpu.SemaphoreType.DMA,
            pltpu.SemaphoreType.DMA,
        ],
    )


def _sc_deg_body(dst2, deg_out, deg_sh, dst_slab, ones_v):
    c = lax.axis_index("c")
    s = lax.axis_index("s")
    w = s * NC + c

    def z_fill(k, _):
        ones_v[k // 8, pl.ds((k % 8) * 16, 16)] = jnp.zeros((16,), jnp.float32)
        return _
    lax.fori_loop(0, IW * 8, z_fill, None)
    for j in range(SUBR // IW):
        pltpu.sync_copy(ones_v, deg_sh.at[pl.ds(s * SUBR + j * IW, IW)])

    def o_fill(k, _):
        ones_v[k // 8, pl.ds((k % 8) * 16, 16)] = jnp.ones((16,), jnp.float32)
        return _
    lax.fori_loop(0, IW * 8, o_fill, None)
    plsc.subcore_barrier()

    for ci in range(NCHUNK):
        pltpu.sync_copy(dst2.at[pl.ds(w * WPB + ci * CHUNK, CHUNK)], dst_slab)

        def step(i, _):
            pltpu.sync_copy(ones_v, deg_sh.at[dst_slab.at[i]], add=True)
            return _
        lax.fori_loop(0, CHUNK, step, None)
    plsc.subcore_barrier()

    pltpu.sync_copy(deg_sh.at[pl.ds(s * SUBR, SUBR)],
                    deg_out.at[c].at[pl.ds(s * SUBR, SUBR)])


@functools.lru_cache(maxsize=None)
def _sc_deg_kernel():
    return pl.kernel(
        _sc_deg_body,
        out_type=jax.ShapeDtypeStruct((NC, ACC_N, DEGW), jnp.float32),
        mesh=plsc.VectorSubcoreMesh(core_axis_name="c", subcore_axis_name="s"),
        scratch_types=[
            pltpu.VMEM_SHARED((ACC_N, DEGW), jnp.float32),
            pltpu.VMEM((CHUNK, IW), jnp.int32),
            pltpu.VMEM((IW, DEGW), jnp.float32),
        ],
    )


BM = 1000


def _tc_compute(h_ref, a0_ref, a1_ref, d0_ref, d1_ref, ws_ref, wn_ref, b_ref):
    deg = jnp.maximum(d0_ref[0, :, 0:1] + d1_ref[0, :, 0:1], 1.0)
    hn = (a0_ref[0] + a1_ref[0]) / deg
    return (
        jnp.dot(h_ref[...], ws_ref[...], preferred_element_type=jnp.float32)
        + jnp.dot(hn, wn_ref[...], preferred_element_type=jnp.float32)
        + b_ref[...])


def _tc_update_body(h_ref, a0_ref, a1_ref, d0_ref, d1_ref, ws_ref, wn_ref,
                    b_ref, o_ref):
    o_ref[...] = _tc_compute(h_ref, a0_ref, a1_ref, d0_ref, d1_ref, ws_ref,
                             wn_ref, b_ref)


def _tc_update_body2(h_ref, a0_ref, a1_ref, d0_ref, d1_ref, ws_ref, wn_ref,
                     b_ref, o_ref, ob_ref):
    res = _tc_compute(h_ref, a0_ref, a1_ref, d0_ref, d1_ref, ws_ref, wn_ref,
                      b_ref)
    o_ref[...] = res
    ob_ref[...] = res.astype(jnp.bfloat16)


def _tc_update(h, agg, deg, Ws, Wn, b, with_bf16: bool):
    out_shape = [jax.ShapeDtypeStruct((N, D), jnp.float32)]
    out_specs = [pl.BlockSpec((BM, D), lambda i: (i, 0))]
    if with_bf16:
        out_shape.append(jax.ShapeDtypeStruct((N, D), jnp.bfloat16))
        out_specs.append(pl.BlockSpec((BM, D), lambda i: (i, 0)))
    return pl.pallas_call(
        _tc_update_body2 if with_bf16 else _tc_update_body,
        grid=(N // BM,),
        in_specs=[
            pl.BlockSpec((BM, D), lambda i: (i, 0)),
            pl.BlockSpec((1, BM, D), lambda i: (0, i, 0)),
            pl.BlockSpec((1, BM, D), lambda i: (1, i, 0)),
            pl.BlockSpec((1, BM, DEGW), lambda i: (0, i, 0)),
            pl.BlockSpec((1, BM, DEGW), lambda i: (1, i, 0)),
            pl.BlockSpec((D, D), lambda i: (0, 0)),
            pl.BlockSpec((D, D), lambda i: (0, 0)),
            pl.BlockSpec((1, D), lambda i: (0, 0)),
        ],
        out_specs=out_specs if with_bf16 else out_specs[0],
        out_shape=out_shape if with_bf16 else out_shape[0],
    )(h, agg, agg, deg, deg, Ws, Wn, b.reshape(1, D))


def _pack_bf16(hb):
    return lax.bitcast_convert_type(hb.reshape(N, DW, 2), jnp.int32)


def kernel(x, edge_index, Ws0, Wn0, b0, Ws1, Wn1, b1, Ws2, Wn2, b2):
    pad = SROWS * IW - E
    srcp = jnp.concatenate(
        [edge_index[0], jnp.zeros((pad,), jnp.int32)]).reshape(SROWS, IW)
    dstp = jnp.concatenate(
        [edge_index[1], jnp.full((pad,), N, jnp.int32)]).reshape(SROWS, IW)
    tix = jnp.asarray(_T_IDX)
    Wn0p, Wn1p, Wn2p = Wn0[tix, :], Wn1[tix, :], Wn2[tix, :]

    deg = _sc_deg_kernel()(dstp)
    agg1 = _sc_agg_kernel()(_pack_bf16(x.astype(jnp.bfloat16)), srcp, dstp)
    h1, h1b = _tc_update(x, agg1, deg, Ws0, Wn0p, b0, True)
    agg2 = _sc_agg_kernel()(_pack_bf16(h1b), srcp, dstp)
    h2, h2b = _tc_update(h1, agg2, deg, Ws1, Wn1p, b1, True)
    agg3 = _sc_agg_kernel()(_pack_bf16(h2b), srcp, dstp)
    return _tc_update(h2, agg3, deg, Ws2, Wn2p, b2, False)


# speedup vs baseline: 1.1968x; 1.1968x over previous
import functools

import jax
import jax.numpy as jnp
from jax import lax
from jax.experimental import pallas as pl
from jax.experimental.pallas import tpu as pltpu
from jax.experimental.pallas import tpu_sc as plsc

N = 10000
E = 320000
D = 128
IW = 64
NC = 2
NS = 16
NW = NC * NS
WPB = 160
SROWS = NW * WPB
WPB0 = 288
WPB1 = 32
CHUNK = 32
SUBR = 640
ACC_N = NS * SUBR
DEGW = 128


def _sc_agg_body(table, src2, dst2, agg_out, acc_sh, src_slab, dst_slab,
                 rows_v0, rows_v1, sem_g0, sem_g1):
    c = lax.axis_index("c")
    s = lax.axis_index("s")

    def z_rows(k, _):
        rows_v0[k // 8, pl.ds((k % 8) * 16, 16)] = jnp.zeros((16,), jnp.float32)
        return _
    lax.fori_loop(0, IW * 8, z_rows, None)
    for j in range(SUBR // IW):
        pltpu.sync_copy(rows_v0, acc_sh.at[pl.ds(s * SUBR + j * IW, IW)])
    plsc.subcore_barrier()

    def gather(r, buf, sem):
        return pltpu.async_copy(table.at[src_slab.at[r]], buf, sem)

    def wait_gather(buf, sem):
        pltpu.make_async_copy(table.at[pl.ds(0, IW)], buf, sem).wait()

    def scatter(r, buf):
        pltpu.sync_copy(buf, acc_sh.at[dst_slab.at[r]], add=True)

    def run_chunks(base_rows, nchunk):
        for ci in range(nchunk):
            pltpu.sync_copy(src2.at[pl.ds(base_rows + ci * CHUNK, CHUNK)],
                            src_slab)
            pltpu.sync_copy(dst2.at[pl.ds(base_rows + ci * CHUNK, CHUNK)],
                            dst_slab)
            gather(0, rows_v0, sem_g0)

            def pair(p, _):
                wait_gather(rows_v0, sem_g0)
                gather(2 * p + 1, rows_v1, sem_g1)
                scatter(2 * p, rows_v0)
                wait_gather(rows_v1, sem_g1)

                @pl.when(p < CHUNK // 2 - 1)
                def _():
                    gather(2 * p + 2, rows_v0, sem_g0)
                scatter(2 * p + 1, rows_v1)
                return _
            lax.fori_loop(0, CHUNK // 2, pair, None)

    @pl.when(c == 0)
    def _():
        run_chunks(s * WPB0, WPB0 // CHUNK)

    if WPB1:
        @pl.when(c == 1)
        def _():
            run_chunks(NS * WPB0 + s * WPB1, WPB1 // CHUNK)
    plsc.subcore_barrier()

    pltpu.sync_copy(acc_sh.at[pl.ds(s * SUBR, SUBR)],
                    agg_out.at[c].at[pl.ds(s * SUBR, SUBR)])


@functools.lru_cache(maxsize=None)
def _sc_agg_kernel():
    return pl.kernel(
        _sc_agg_body,
        out_type=jax.ShapeDtypeStruct((NC, ACC_N, D), jnp.float32),
        mesh=plsc.VectorSubcoreMesh(core_axis_name="c", subcore_axis_name="s"),
        scratch_types=[
            pltpu.VMEM_SHARED((ACC_N, D), jnp.float32),
            pltpu.VMEM((CHUNK, IW), jnp.int32),
            pltpu.VMEM((CHUNK, IW), jnp.int32),
            pltpu.VMEM((IW, D), jnp.float32),
            pltpu.VMEM((IW, D), jnp.float32),
            pltpu.SemaphoreType.DMA,
            pltpu.SemaphoreType.DMA,
        ],
    )


def _sc_deg_body(dst2, deg_out, deg_sh, dst_slab, ones_v):
    c = lax.axis_index("c")
    s = lax.axis_index("s")
    w = s * NC + c

    def z_fill(k, _):
        ones_v[k // 8, pl.ds((k % 8) * 16, 16)] = jnp.zeros((16,), jnp.float32)
        return _
    lax.fori_loop(0, IW * 8, z_fill, None)
    for j in range(SUBR // IW):
        pltpu.sync_copy(ones_v, deg_sh.at[pl.ds(s * SUBR + j * IW, IW)])

    def o_fill(k, _):
        ones_v[k // 8, pl.ds((k % 8) * 16, 16)] = jnp.ones((16,), jnp.float32)
        return _
    lax.fori_loop(0, IW * 8, o_fill, None)
    plsc.subcore_barrier()

    for ci in range(WPB // CHUNK):
        pltpu.sync_copy(dst2.at[pl.ds(w * WPB + ci * CHUNK, CHUNK)], dst_slab)

        def step(i, _):
            pltpu.sync_copy(ones_v, deg_sh.at[dst_slab.at[i]], add=True)
            return _
        lax.fori_loop(0, CHUNK, step, None)
    plsc.subcore_barrier()

    pltpu.sync_copy(deg_sh.at[pl.ds(s * SUBR, SUBR)],
                    deg_out.at[c].at[pl.ds(s * SUBR, SUBR)])


@functools.lru_cache(maxsize=None)
def _sc_deg_kernel():
    return pl.kernel(
        _sc_deg_body,
        out_type=jax.ShapeDtypeStruct((NC, ACC_N, DEGW), jnp.float32),
        mesh=plsc.VectorSubcoreMesh(core_axis_name="c", subcore_axis_name="s"),
        scratch_types=[
            pltpu.VMEM_SHARED((ACC_N, DEGW), jnp.float32),
            pltpu.VMEM((CHUNK, IW), jnp.int32),
            pltpu.VMEM((IW, DEGW), jnp.float32),
        ],
    )


BM = 1000


def _tc_update_body(h_ref, a0_ref, a1_ref, d0_ref, d1_ref, ws_ref, wn_ref,
                    b_ref, o_ref):
    deg = jnp.maximum(d0_ref[0, :, 0:1] + d1_ref[0, :, 0:1], 1.0)
    hn = (a0_ref[0] + a1_ref[0]) / deg
    o_ref[...] = (
        jnp.dot(h_ref[...], ws_ref[...], preferred_element_type=jnp.float32)
        + jnp.dot(hn, wn_ref[...], preferred_element_type=jnp.float32)
        + b_ref[...])


def _tc_update(h, agg, deg, Ws, Wn, b):
    return pl.pallas_call(
        _tc_update_body,
        grid=(N // BM,),
        in_specs=[
            pl.BlockSpec((BM, D), lambda i: (i, 0)),
            pl.BlockSpec((1, BM, D), lambda i: (0, i, 0)),
            pl.BlockSpec((1, BM, D), lambda i: (1, i, 0)),
            pl.BlockSpec((1, BM, DEGW), lambda i: (0, i, 0)),
            pl.BlockSpec((1, BM, DEGW), lambda i: (1, i, 0)),
            pl.BlockSpec((D, D), lambda i: (0, 0)),
            pl.BlockSpec((D, D), lambda i: (0, 0)),
            pl.BlockSpec((1, D), lambda i: (0, 0)),
        ],
        out_specs=pl.BlockSpec((BM, D), lambda i: (i, 0)),
        out_shape=jax.ShapeDtypeStruct((N, D), jnp.float32),
    )(h, agg, agg, deg, deg, Ws, Wn, b.reshape(1, D))


def kernel(x, edge_index, Ws0, Wn0, b0, Ws1, Wn1, b1, Ws2, Wn2, b2):
    pad = SROWS * IW - E
    srcp = jnp.concatenate(
        [edge_index[0], jnp.zeros((pad,), jnp.int32)]).reshape(SROWS, IW)
    dstp = jnp.concatenate(
        [edge_index[1], jnp.full((pad,), N, jnp.int32)]).reshape(SROWS, IW)

    deg = _sc_deg_kernel()(dstp)
    agg1 = _sc_agg_kernel()(x, srcp, dstp)
    h1 = _tc_update(x, agg1, deg, Ws0, Wn0, b0)
    agg2 = _sc_agg_kernel()(h1, srcp, dstp)
    h2 = _tc_update(h1, agg2, deg, Ws1, Wn1, b1)
    agg3 = _sc_agg_kernel()(h2, srcp, dstp)
    return _tc_update(h2, agg3, deg, Ws2, Wn2, b2)
